# speedup vs baseline: 6.3178x; 6.3178x over previous
import functools

import jax
import jax.numpy as jnp
from jax import lax
from jax.experimental import pallas as pl
from jax.experimental.pallas import tpu as pltpu
from jax.experimental.pallas import tpu_sc as plsc

_NUM_WORKERS = 32
_CHUNK = 128


def kernel(x, W):
    B, H = x.shape
    V, D = W.shape
    N = B * H
    per_w = N // _NUM_WORKERS
    n_chunks = per_w // _CHUNK
    assert per_w * _NUM_WORKERS == N and n_chunks * _CHUNK == per_w

    xf = x.reshape(_NUM_WORKERS, n_chunks, _CHUNK).astype(jnp.int32)
    mesh = plsc.VectorSubcoreMesh(core_axis_name="c", subcore_axis_name="s")

    @functools.partial(
        pl.kernel,
        mesh=mesh,
        out_type=jax.ShapeDtypeStruct((N, D), jnp.float32),
        scratch_types=[
            pltpu.VMEM((n_chunks, _CHUNK), jnp.int32),
            pltpu.VMEM((_CHUNK, D), jnp.float32),
            pltpu.SemaphoreType.DMA,
        ],
    )
    def _embed(x_hbm, w_hbm, out_hbm, idx_v, buf, gsem):
        wid = lax.axis_index("s") * 2 + lax.axis_index("c")
        base = wid * per_w
        pltpu.sync_copy(x_hbm.at[wid], idx_v)

        def body(c, _):
            pltpu.async_copy(w_hbm.at[idx_v.at[c]], buf, gsem).wait()
            pltpu.sync_copy(buf, out_hbm.at[pl.ds(base + c * _CHUNK, _CHUNK)])
            return ()

        lax.fori_loop(0, n_chunks, body, ())

    out = _embed(xf, W)
    return out.reshape(B, H, D)


# speedup vs baseline: 9.1599x; 1.4499x over previous
import functools

import jax
import jax.numpy as jnp
from jax import lax
from jax.experimental import pallas as pl
from jax.experimental.pallas import tpu as pltpu
from jax.experimental.pallas import tpu_sc as plsc

_NUM_WORKERS = 32
_CHUNK = 128
_NBUF = 4


def kernel(x, W):
    B, H = x.shape
    V, D = W.shape
    N = B * H
    per_w = N // _NUM_WORKERS
    n_chunks = per_w // _CHUNK
    assert per_w * _NUM_WORKERS == N and n_chunks * _CHUNK == per_w
    assert n_chunks % _NBUF == 0 and n_chunks >= 2 * _NBUF

    xf = x.reshape(_NUM_WORKERS, n_chunks, _CHUNK).astype(jnp.int32)
    mesh = plsc.VectorSubcoreMesh(core_axis_name="c", subcore_axis_name="s")

    @functools.partial(
        pl.kernel,
        mesh=mesh,
        out_type=jax.ShapeDtypeStruct((N, D), jnp.float32),
        scratch_types=(
            [pltpu.VMEM((n_chunks, _CHUNK), jnp.int32)]
            + [pltpu.VMEM((_CHUNK, D), jnp.float32)] * _NBUF
            + [pltpu.SemaphoreType.DMA] * (2 * _NBUF)
        ),
    )
    def _embed(x_hbm, w_hbm, out_hbm, idx_v, *rest):
        bufs = rest[:_NBUF]
        gsems = rest[_NBUF:2 * _NBUF]
        ssems = rest[2 * _NBUF:]
        wid = lax.axis_index("s") * 2 + lax.axis_index("c")
        base = wid * per_w
        pltpu.sync_copy(x_hbm.at[wid], idx_v)

        def gather(c, b):
            return pltpu.make_async_copy(
                w_hbm.at[idx_v.at[c]], bufs[b], gsems[b])

        def scatter(c, b):
            return pltpu.make_async_copy(
                bufs[b], out_hbm.at[pl.ds(base + c * _CHUNK, _CHUNK)],
                ssems[b])

        gather(0, 0).start()
        gather(1, 1).start()

        def body(g, _):
            for b in range(_NBUF):
                c = g * _NBUF + b
                gather(c, b).wait()
                scatter(c, b).start()
                cn = c + 2
                bn = (b + 2) % _NBUF

                @pl.when(cn < n_chunks)
                def _issue():
                    @pl.when(c >= 2)
                    def _drain():
                        scatter(c - 2, bn).wait()
                    gather(cn, bn).start()
            return ()

        lax.fori_loop(0, n_chunks // _NBUF, body, ())
        for b in range(_NBUF):
            scatter(n_chunks - _NBUF + b, b).wait()

    out = _embed(xf, W)
    return out.reshape(B, H, D)


# speedup vs baseline: 9.1729x; 1.0014x over previous
import functools

import jax
import jax.numpy as jnp
from jax import lax
from jax.experimental import pallas as pl
from jax.experimental.pallas import tpu as pltpu
from jax.experimental.pallas import tpu_sc as plsc

_NUM_WORKERS = 32
_CHUNK = 128
_NBUF = 5
_LOOK = 3


def kernel(x, W):
    B, H = x.shape
    V, D = W.shape
    N = B * H
    per_w = N // _NUM_WORKERS
    n_chunks = per_w // _CHUNK
    assert per_w * _NUM_WORKERS == N and n_chunks * _CHUNK == per_w
    assert n_chunks % _NBUF == 0 and n_chunks >= 2 * _NBUF

    xf = x.reshape(_NUM_WORKERS, n_chunks, _CHUNK).astype(jnp.int32)
    mesh = plsc.VectorSubcoreMesh(core_axis_name="c", subcore_axis_name="s")

    @functools.partial(
        pl.kernel,
        mesh=mesh,
        out_type=jax.ShapeDtypeStruct((N, D), jnp.float32),
        scratch_types=(
            [pltpu.VMEM((n_chunks, _CHUNK), jnp.int32)]
            + [pltpu.VMEM((_CHUNK, D), jnp.float32)] * _NBUF
            + [pltpu.SemaphoreType.DMA] * (2 * _NBUF)
        ),
    )
    def _embed(x_hbm, w_hbm, out_hbm, idx_v, *rest):
        bufs = rest[:_NBUF]
        gsems = rest[_NBUF:2 * _NBUF]
        ssems = rest[2 * _NBUF:]
        wid = lax.axis_index("s") * 2 + lax.axis_index("c")
        base = wid * per_w
        pltpu.sync_copy(x_hbm.at[wid], idx_v)

        def gather(c, b):
            return pltpu.make_async_copy(
                w_hbm.at[idx_v.at[c]], bufs[b], gsems[b])

        def scatter(c, b):
            return pltpu.make_async_copy(
                bufs[b], out_hbm.at[pl.ds(base + c * _CHUNK, _CHUNK)],
                ssems[b])

        for b in range(_LOOK):
            gather(b, b).start()

        def body(g, _):
            for b in range(_NBUF):
                c = g * _NBUF + b
                gather(c, b).wait()
                scatter(c, b).start()
                cn = c + _LOOK
                bn = (b + _LOOK) % _NBUF

                @pl.when(cn < n_chunks)
                def _issue():
                    @pl.when(c >= _NBUF - _LOOK)
                    def _drain():
                        scatter(c - (_NBUF - _LOOK), bn).wait()
                    gather(cn, bn).start()
            return ()

        lax.fori_loop(0, n_chunks // _NBUF, body, ())
        for b in range(_NBUF):
            scatter(n_chunks - _NBUF + b, b).wait()

    out = _embed(xf, W)
    return out.reshape(B, H, D)


# speedup vs baseline: 9.1984x; 1.0028x over previous
import functools

import jax
import jax.numpy as jnp
from jax import lax
from jax.experimental import pallas as pl
from jax.experimental.pallas import tpu as pltpu
from jax.experimental.pallas import tpu_sc as plsc

_NUM_WORKERS = 32
_CHUNK = 128
_CH = 3


def kernel(x, W):
    B, H = x.shape
    V, D = W.shape
    N = B * H
    per_w = N // _NUM_WORKERS
    n_chunks = per_w // _CHUNK
    assert per_w * _NUM_WORKERS == N and n_chunks * _CHUNK == per_w
    full_halves = n_chunks // _CH
    tail = n_chunks - full_halves * _CH
    assert full_halves % 2 == 0 and full_halves >= 4

    xf = x.reshape(_NUM_WORKERS, n_chunks, _CHUNK).astype(jnp.int32)
    mesh = plsc.VectorSubcoreMesh(core_axis_name="c", subcore_axis_name="s")

    @functools.partial(
        pl.kernel,
        mesh=mesh,
        out_type=jax.ShapeDtypeStruct((N, D), jnp.float32),
        scratch_types=(
            [pltpu.VMEM((n_chunks, _CHUNK), jnp.int32)]
            + [pltpu.VMEM((_CH * _CHUNK, D), jnp.float32)] * 2
            + [pltpu.SemaphoreType.DMA] * 4
        ),
    )
    def _embed(x_hbm, w_hbm, out_hbm, idx_v, hb0, hb1, g0, g1, s0, s1):
        hb = (hb0, hb1)
        gsems = (g0, g1)
        ssems = (s0, s1)
        wid = lax.axis_index("s") * 2 + lax.axis_index("c")
        base = wid * per_w
        pltpu.sync_copy(x_hbm.at[wid], idx_v)

        def gath(c, p, k):
            return pltpu.make_async_copy(
                w_hbm.at[idx_v.at[c]],
                hb[p].at[pl.ds(k * _CHUNK, _CHUNK)], gsems[p])

        def scat(h, p, nch):
            return pltpu.make_async_copy(
                hb[p].at[pl.ds(0, nch * _CHUNK)],
                out_hbm.at[pl.ds(base + h * _CH * _CHUNK, nch * _CHUNK)],
                ssems[p])

        for k in range(_CH):
            gath(k, 0, k).start()

        def body(g, _):
            for p in range(2):
                h = g * 2 + p
                pn = 1 - p

                @pl.when(h >= 1)
                def _drain():
                    scat(h - 1, pn, _CH).wait()

                @pl.when(h + 1 < full_halves)
                def _issue():
                    for k in range(_CH):
                        gath((h + 1) * _CH + k, pn, k).start()

                for k in range(_CH):
                    gath(h * _CH + k, p, k).wait()
                scat(h, p, _CH).start()
            return ()

        lax.fori_loop(0, full_halves // 2, body, ())

        if tail:
            for k in range(tail):
                gath(full_halves * _CH + k, 0, k).start()
            for k in range(tail):
                gath(full_halves * _CH + k, 0, k).wait()
            scat(full_halves, 0, tail).start()
            scat(full_halves, 0, tail).wait()
        scat(full_halves - 1, 1, _CH).wait()

    out = _embed(xf, W)
    return out.reshape(B, H, D)
